# Initial kernel scaffold; baseline (speedup 1.0000x reference)
import jax
import jax.numpy as jnp
from jax.experimental import pallas as pl


def kernel(x, up_x, unq_inv):
    raise NotImplementedError("write your pallas kernel here")


# speedup vs baseline: 2.0451x; 2.0451x over previous
import functools

import jax
import jax.numpy as jnp
from jax import lax
from jax.experimental import pallas as pl
from jax.experimental.pallas import tpu as pltpu
from jax.experimental.pallas import tpu_sc as plsc

_LANES = 16
_B = 128


def _body(x_hbm, upx_hbm, idx_hbm, out_hbm, idx_v, gath_v, upx_v, sem,
          *, m, num_chunks, chunks_per_worker, num_workers):
    wid = lax.axis_index("s") * 2 + lax.axis_index("c")
    start = wid * chunks_per_worker
    end = jnp.minimum(start + chunks_per_worker, num_chunks)

    @pl.loop(start, end)
    def _chunk(i):
        off = jnp.minimum(i * _B, m - _B)
        pltpu.sync_copy(idx_hbm.at[pl.ds(off, _B)], idx_v)
        cp = pltpu.async_copy(x_hbm.at[idx_v], gath_v, sem)
        pltpu.sync_copy(upx_hbm.at[pl.ds(off, _B)], upx_v)
        cp.wait()

        @pl.loop(0, _B)
        def _row(r):
            for j in range(128 // _LANES):
                c = j * _LANES
                upx_v[r, pl.ds(c, _LANES)] = (
                    upx_v[r, pl.ds(c, _LANES)] + gath_v[r, pl.ds(c, _LANES)]
                )

        pltpu.sync_copy(upx_v, out_hbm.at[pl.ds(off, _B)])


def kernel(x, up_x, unq_inv):
    m, c = up_x.shape
    idx = unq_inv.astype(jnp.int32)
    num_chunks = (m + _B - 1) // _B
    num_workers = 32
    chunks_per_worker = (num_chunks + num_workers - 1) // num_workers

    mesh = plsc.VectorSubcoreMesh(core_axis_name="c", subcore_axis_name="s")
    body = functools.partial(
        _body,
        m=m,
        num_chunks=num_chunks,
        chunks_per_worker=chunks_per_worker,
        num_workers=num_workers,
    )
    run = pl.kernel(
        body,
        out_type=jax.ShapeDtypeStruct((m, c), jnp.float32),
        mesh=mesh,
        scratch_types=[
            pltpu.VMEM((_B,), jnp.int32),
            pltpu.VMEM((_B, c), jnp.float32),
            pltpu.VMEM((_B, c), jnp.float32),
            pltpu.SemaphoreType.DMA,
        ],
    )
    return run(x, up_x, idx)
